# baseline (speedup 1.0000x reference)
import functools

import jax
import jax.numpy as jnp
from jax import lax
from jax.experimental import pallas as pl
from jax.experimental.pallas import tpu as pltpu
from jax.experimental.pallas import tpu_sc as plsc

B = 4096
D = 16
P1 = 64
P2 = 32
EPS = 1e-5

_NC = 2
_NS = 16
_NW = _NC * _NS
_BPW = B // _NW


def _gather_body(uid, iid, a1id, a2id, t_u, t_i, t_a1, t_a2,
                 out_u, out_i, out_a1, out_a2,
                 idx_v, rows_v, sem):
    wid = lax.axis_index("s") * _NC + lax.axis_index("c")
    base = wid * _BPW
    for idx_hbm, tbl, out in ((uid, t_u, out_u), (iid, t_i, out_i),
                              (a1id, t_a1, out_a1), (a2id, t_a2, out_a2)):
        pltpu.sync_copy(idx_hbm.at[pl.ds(base, _BPW)], idx_v)
        pltpu.async_copy(tbl.at[idx_v], rows_v, sem).wait()
        pltpu.sync_copy(rows_v, out.at[pl.ds(base, _BPW)])


@functools.cache
def _gather4():
    return functools.partial(
        pl.kernel,
        mesh=plsc.VectorSubcoreMesh(core_axis_name="c", subcore_axis_name="s"),
        out_type=[jax.ShapeDtypeStruct((B, D), jnp.float32)] * 4,
        scratch_types=[
            pltpu.VMEM((_BPW,), jnp.int32),
            pltpu.VMEM((_BPW, D), jnp.float32),
            pltpu.SemaphoreType.DMA,
        ],
        compiler_params=pltpu.CompilerParams(use_tc_tiling_on_sc=False),
    )(_gather_body)


def _mlp_body(u_ref, i_ref, a1_ref, a2_ref,
              W1_ref, b1_ref, g1_ref, be1_ref,
              W2_ref, b2_ref, g2_ref, be2_ref,
              W3_ref, b3_ref, out_ref):
    x = jnp.concatenate(
        [u_ref[...], i_ref[...], a1_ref[...], a2_ref[...]], axis=1)
    h = jnp.dot(x, W1_ref[...].T, preferred_element_type=jnp.float32)
    h = h + b1_ref[...]
    m = jnp.mean(h, axis=0, keepdims=True)
    v = jnp.mean((h - m) ** 2, axis=0, keepdims=True)
    h = (h - m) * lax.rsqrt(v + EPS) * g1_ref[...] + be1_ref[...]
    h = jnp.maximum(h, 0.0)
    h = jnp.dot(h, W2_ref[...].T, preferred_element_type=jnp.float32)
    h = h + b2_ref[...]
    m = jnp.mean(h, axis=0, keepdims=True)
    v = jnp.mean((h - m) ** 2, axis=0, keepdims=True)
    h = (h - m) * lax.rsqrt(v + EPS) * g2_ref[...] + be2_ref[...]
    h = jnp.maximum(h, 0.0)
    out_ref[...] = (jnp.sum(h * W3_ref[...], axis=1, keepdims=True)
                    + b3_ref[...])


def _mlp(u, it, a1, a2, W1, b1, g1, be1, W2, b2, g2, be2, W3, b3):
    return pl.pallas_call(
        _mlp_body,
        out_shape=jax.ShapeDtypeStruct((B, 1), jnp.float32),
    )(u, it, a1, a2, W1, b1, g1, be1, W2, b2, g2, be2, W3, b3)


def kernel(user_id, item_id, attr1_id, attr2_id,
           emb_user, emb_item, emb_attr1, emb_attr2,
           W1, b1, g1, be1, W2, b2, g2, be2, W3, b3):
    uid = user_id.astype(jnp.int32)
    iid = item_id.astype(jnp.int32)
    a1id = attr1_id.astype(jnp.int32)
    a2id = attr2_id.astype(jnp.int32)
    u, it, a1, a2 = _gather4()(uid, iid, a1id, a2id,
                               emb_user, emb_item, emb_attr1, emb_attr2)
    return _mlp(u, it, a1, a2, W1, b1, g1, be1, W2, b2, g2, be2, W3, b3)


# speedup vs baseline: 1.5095x; 1.5095x over previous
import functools

import jax
import jax.numpy as jnp
from jax import lax
from jax.experimental import pallas as pl
from jax.experimental.pallas import tpu as pltpu
from jax.experimental.pallas import tpu_sc as plsc

B = 4096
D = 16
P1 = 64
P2 = 32
EPS = 1e-5

_NC = 2
_NS = 16
_NW = _NC * _NS
_BPW = B // _NW


def _gather_body(uid, iid, a1id, a2id, t_u, t_i, t_a1, t_a2,
                 out_u, out_i, out_a1, out_a2,
                 i0, i1, i2, i3, r0, r1, r2, r3, sem):
    wid = lax.axis_index("s") * _NC + lax.axis_index("c")
    base = wid * _BPW
    tabs = ((uid, t_u, out_u, i0, r0), (iid, t_i, out_i, i1, r1),
            (a1id, t_a1, out_a1, i2, r2), (a2id, t_a2, out_a2, i3, r3))
    for idx_hbm, _, _, idx_v, _ in tabs:
        pltpu.sync_copy(idx_hbm.at[pl.ds(base, _BPW)], idx_v)
    for _, tbl, _, idx_v, rows_v in tabs:
        def body(g, _, tbl=tbl, idx_v=idx_v, rows_v=rows_v):
            v = idx_v[pl.ds(g * 16, 16)]
            for l in range(16):
                pltpu.make_async_copy(
                    tbl.at[v[l]], rows_v.at[g * 16 + l], sem).start()
            return 0
        lax.fori_loop(0, _BPW // 16, body, 0)
    for _, tbl, _, _, rows_v in tabs:
        pltpu.make_async_copy(tbl.at[pl.ds(0, _BPW)], rows_v, sem).wait()
    for _, _, out, _, rows_v in tabs:
        pltpu.sync_copy(rows_v, out.at[pl.ds(base, _BPW)])


@functools.cache
def _gather4():
    return functools.partial(
        pl.kernel,
        mesh=plsc.VectorSubcoreMesh(core_axis_name="c", subcore_axis_name="s"),
        out_type=[jax.ShapeDtypeStruct((B, D), jnp.float32)] * 4,
        scratch_types=[pltpu.VMEM((_BPW,), jnp.int32)] * 4
        + [pltpu.VMEM((_BPW, D), jnp.float32)] * 4
        + [pltpu.SemaphoreType.DMA],
    )(_gather_body)


def _mlp_body(u_ref, i_ref, a1_ref, a2_ref,
              W1_ref, b1_ref, g1_ref, be1_ref,
              W2_ref, b2_ref, g2_ref, be2_ref,
              W3_ref, b3_ref, out_ref):
    x = jnp.concatenate(
        [u_ref[...], i_ref[...], a1_ref[...], a2_ref[...]], axis=1)
    h = jnp.dot(x, W1_ref[...].T, preferred_element_type=jnp.float32)
    h = h + b1_ref[...]
    m = jnp.mean(h, axis=0, keepdims=True)
    v = jnp.mean((h - m) ** 2, axis=0, keepdims=True)
    h = (h - m) * lax.rsqrt(v + EPS) * g1_ref[...] + be1_ref[...]
    h = jnp.maximum(h, 0.0)
    h = jnp.dot(h, W2_ref[...].T, preferred_element_type=jnp.float32)
    h = h + b2_ref[...]
    m = jnp.mean(h, axis=0, keepdims=True)
    v = jnp.mean((h - m) ** 2, axis=0, keepdims=True)
    h = (h - m) * lax.rsqrt(v + EPS) * g2_ref[...] + be2_ref[...]
    h = jnp.maximum(h, 0.0)
    out_ref[...] = (jnp.sum(h * W3_ref[...], axis=1, keepdims=True)
                    + b3_ref[...])


def _mlp(u, it, a1, a2, W1, b1, g1, be1, W2, b2, g2, be2, W3, b3):
    return pl.pallas_call(
        _mlp_body,
        out_shape=jax.ShapeDtypeStruct((B, 1), jnp.float32),
    )(u, it, a1, a2, W1, b1, g1, be1, W2, b2, g2, be2, W3, b3)


def kernel(user_id, item_id, attr1_id, attr2_id,
           emb_user, emb_item, emb_attr1, emb_attr2,
           W1, b1, g1, be1, W2, b2, g2, be2, W3, b3):
    uid = user_id.astype(jnp.int32)
    iid = item_id.astype(jnp.int32)
    a1id = attr1_id.astype(jnp.int32)
    a2id = attr2_id.astype(jnp.int32)
    u, it, a1, a2 = _gather4()(uid, iid, a1id, a2id,
                               emb_user, emb_item, emb_attr1, emb_attr2)
    return _mlp(u, it, a1, a2, W1, b1, g1, be1, W2, b2, g2, be2, W3, b3)
